# Initial kernel scaffold; baseline (speedup 1.0000x reference)
import jax
import jax.numpy as jnp
from jax.experimental import pallas as pl


def kernel(features, neighbor_idx, W1, gamma1, beta1, W2, gamma2, beta2):
    raise NotImplementedError("write your pallas kernel here")


# speedup vs baseline: 20.4526x; 20.4526x over previous
import functools

import jax
import jax.numpy as jnp
from jax import lax
from jax.experimental import pallas as pl
from jax.experimental.pallas import tpu as pltpu
from jax.experimental.pallas import tpu_sc as plsc

N, C, K = 200000, 16, 27
EPS = 1e-5

NW = 32
PER_W = 6272
N2 = NW * PER_W
CH = 128
CHUNKS = PER_W // CH
ROWS = CH * K
STRIPS = ROWS // 128

BLK = 1024
FBLK = 800


def _embed_sum_body(table_hbm, cidx_hbm, h_hbm, stats_hbm,
                    idx_v, rows_v, out_v, stats_v, isem, gsem):
    wid = lax.axis_index("s") * 2 + lax.axis_index("c")
    base = wid * PER_W

    def chunk_body(j, carry):
        s_acc, q_acc = carry
        cbase = base + j * CH
        ioff = pl.multiple_of(cbase * K, 8)
        pltpu.async_copy(cidx_hbm.at[pl.ds(ioff, ROWS)], idx_v, isem).wait()
        copies = []
        for s in range(STRIPS):
            copies.append(pltpu.async_copy(
                table_hbm.at[idx_v.at[pl.ds(s * 128, 128)]],
                rows_v.at[pl.ds(s * 128, 128)], gsem))
        for cp in copies:
            cp.wait()

        def voxel_body(v, carry2):
            s2, q2 = carry2
            r0 = v * K
            acc = rows_v[r0, :]
            for k in range(1, K):
                acc = acc + rows_v[r0 + k, :]
            out_v[v, :] = acc
            m = jnp.where(cbase + v < N, 1.0, 0.0)
            return (s2 + acc * m, q2 + acc * acc * m)

        s_acc, q_acc = lax.fori_loop(0, CH, voxel_body, (s_acc, q_acc))
        pltpu.sync_copy(out_v, h_hbm.at[pl.ds(cbase, CH)])
        return (s_acc, q_acc)

    zero = jnp.zeros((C,), jnp.float32)
    s_acc, q_acc = lax.fori_loop(0, CHUNKS, chunk_body, (zero, zero))
    stats_v[0, :] = s_acc
    stats_v[1, :] = q_acc
    pltpu.sync_copy(stats_v, stats_hbm.at[wid])


_embed_sum = functools.partial(
    pl.kernel,
    mesh=plsc.VectorSubcoreMesh(core_axis_name="c", subcore_axis_name="s"),
    out_type=[jax.ShapeDtypeStruct((N2, C), jnp.float32),
              jax.ShapeDtypeStruct((NW, 2, C), jnp.float32)],
    scratch_types=[
        pltpu.VMEM((ROWS,), jnp.int32),
        pltpu.VMEM((ROWS, C), jnp.float32),
        pltpu.VMEM((CH, C), jnp.float32),
        pltpu.VMEM((2, C), jnp.float32),
        pltpu.SemaphoreType.DMA,
        pltpu.SemaphoreType.DMA,
    ],
    compiler_params=pltpu.CompilerParams(use_tc_tiling_on_sc=False),
)(_embed_sum_body)


def _pmat_plain_body(x_ref, w_ref, o_ref):
    o_ref[...] = jnp.dot(x_ref[...], w_ref[...],
                         preferred_element_type=jnp.float32)


def _bn_coeffs(st_ref, g_ref, b_ref):
    st = st_ref[...]
    mu = jnp.sum(st[:, 0, :], axis=0) * (1.0 / N)
    msq = jnp.sum(st[:, 1, :], axis=0) * (1.0 / N)
    var = msq - mu * mu
    a = g_ref[0, :] * lax.rsqrt(var + EPS)
    b = b_ref[0, :] - mu * a
    return a, b


def _pmat_bn_body(x_ref, st_ref, g_ref, b_ref, w_ref, o_ref):
    a, b = _bn_coeffs(st_ref, g_ref, b_ref)
    act = jnp.maximum(x_ref[...] * a[None, :] + b[None, :], 0.0)
    o_ref[...] = jnp.dot(act, w_ref[...], preferred_element_type=jnp.float32)


def _final_body(x_ref, st_ref, g_ref, b_ref, f_ref, o_ref):
    a, b = _bn_coeffs(st_ref, g_ref, b_ref)
    o_ref[...] = jnp.maximum(
        x_ref[...] * a[None, :] + b[None, :] + f_ref[...], 0.0)


def _pmat_plain(x, wcat):
    return pl.pallas_call(
        _pmat_plain_body,
        grid=(N2 // BLK,),
        in_specs=[pl.BlockSpec((BLK, C), lambda i: (i, 0)),
                  pl.BlockSpec((C, K * C), lambda i: (0, 0))],
        out_specs=pl.BlockSpec((BLK, K * C), lambda i: (i, 0)),
        out_shape=jax.ShapeDtypeStruct((N2, K * C), jnp.float32),
    )(x, wcat)


def _pmat_bn(x, st, g, b, wcat):
    return pl.pallas_call(
        _pmat_bn_body,
        grid=(N2 // BLK,),
        in_specs=[pl.BlockSpec((BLK, C), lambda i: (i, 0)),
                  pl.BlockSpec((NW, 2, C), lambda i: (0, 0, 0)),
                  pl.BlockSpec((1, C), lambda i: (0, 0)),
                  pl.BlockSpec((1, C), lambda i: (0, 0)),
                  pl.BlockSpec((C, K * C), lambda i: (0, 0))],
        out_specs=pl.BlockSpec((BLK, K * C), lambda i: (i, 0)),
        out_shape=jax.ShapeDtypeStruct((N2, K * C), jnp.float32),
    )(x, st, g, b, wcat)


def _final(x, st, g, b, feat):
    return pl.pallas_call(
        _final_body,
        grid=(N // FBLK,),
        in_specs=[pl.BlockSpec((FBLK, C), lambda i: (i, 0)),
                  pl.BlockSpec((NW, 2, C), lambda i: (0, 0, 0)),
                  pl.BlockSpec((1, C), lambda i: (0, 0)),
                  pl.BlockSpec((1, C), lambda i: (0, 0)),
                  pl.BlockSpec((FBLK, C), lambda i: (i, 0))],
        out_specs=pl.BlockSpec((FBLK, C), lambda i: (i, 0)),
        out_shape=jax.ShapeDtypeStruct((N, C), jnp.float32),
    )(x, st, g, b, feat)


def kernel(features, neighbor_idx, W1, gamma1, beta1, W2, gamma2, beta2):
    feat_p = jnp.pad(features, ((0, N2 - N), (0, 0)))
    nidx_p = jnp.pad(neighbor_idx, ((0, N2 - N), (0, 0)))
    cidx = (nidx_p * K
            + jnp.arange(K, dtype=jnp.int32)[None, :]).reshape(N2 * K)
    w1cat = W1.transpose(1, 0, 2).reshape(C, K * C)
    w2cat = W2.transpose(1, 0, 2).reshape(C, K * C)
    g1 = gamma1.reshape(1, C)
    b1 = beta1.reshape(1, C)
    g2 = gamma2.reshape(1, C)
    b2 = beta2.reshape(1, C)

    t1 = _pmat_plain(feat_p, w1cat)
    h1, st1 = _embed_sum(t1.reshape(N2 * K, C), cidx)
    t2 = _pmat_bn(h1, st1, g1, b1, w2cat)
    h2, st2 = _embed_sum(t2.reshape(N2 * K, C), cidx)
    return _final(h2, st2, g2, b2, features)
